# Initial kernel scaffold; baseline (speedup 1.0000x reference)
import jax
import jax.numpy as jnp
from jax.experimental import pallas as pl


def kernel(x, table_real, table_imag):
    raise NotImplementedError("write your pallas kernel here")


# speedup vs baseline: 1.3478x; 1.3478x over previous
import functools

import jax
import jax.numpy as jnp
from jax import lax
from jax.experimental import pallas as pl
from jax.experimental.pallas import tpu as pltpu
from jax.experimental.pallas import tpu_sc as plsc

VOCAB = 100000
D = 64
BATCH = 4096
SEQ = 50

_INFO = plsc.get_sparse_core_info()
NC = _INFO.num_cores
NS = _INFO.num_subcores
NW = NC * NS

B = BATCH * SEQ
CHUNK = 128
PER_W = B // NW
N_CHUNKS = PER_W // CHUNK


def _sc_gather(x1d, tab):
    mesh = plsc.VectorSubcoreMesh(core_axis_name="c", subcore_axis_name="s")

    @functools.partial(
        pl.kernel,
        mesh=mesh,
        out_type=jax.ShapeDtypeStruct((B, 2 * D), jnp.float32),
        scratch_types=[
            pltpu.VMEM((PER_W,), jnp.int32),
            pltpu.VMEM((CHUNK, 2 * D), jnp.float32),
            pltpu.SemaphoreType.DMA,
        ],
    )
    def k(x_hbm, tab_hbm, out_hbm, idx_v, buf, sem):
        wid = lax.axis_index("s") * NC + lax.axis_index("c")
        pltpu.sync_copy(x_hbm.at[pl.ds(wid * PER_W, PER_W)], idx_v)

        def body(j, _):
            idx_row = idx_v.at[pl.ds(j * CHUNK, CHUNK)]
            base = wid * PER_W + j * CHUNK
            pltpu.async_copy(tab_hbm.at[idx_row], buf, sem).wait()
            pltpu.sync_copy(buf, out_hbm.at[pl.ds(base, CHUNK)])
            return _

        lax.fori_loop(0, N_CHUNKS, body, 0, unroll=False)

    return k(x1d, tab)


def kernel(x, table_real, table_imag):
    x1d = x.reshape(B).astype(jnp.int32)
    tab = jnp.concatenate([table_real, table_imag], axis=1)
    out = _sc_gather(x1d, tab)
    out_c = lax.complex(out[:, :D], out[:, D:])
    return out_c.reshape(BATCH, SEQ, D)


# speedup vs baseline: 1.3677x; 1.0148x over previous
import functools

import jax
import jax.numpy as jnp
from jax import lax
from jax.experimental import pallas as pl
from jax.experimental.pallas import tpu as pltpu
from jax.experimental.pallas import tpu_sc as plsc

VOCAB = 100000
D = 64
BATCH = 4096
SEQ = 50

_INFO = plsc.get_sparse_core_info()
NC = _INFO.num_cores
NS = _INFO.num_subcores
NW = NC * NS

B = BATCH * SEQ
CHUNK = 128
PER_W = B // NW
N_CHUNKS = PER_W // CHUNK


def _sc_gather(x1d, table_real, table_imag):
    mesh = plsc.VectorSubcoreMesh(core_axis_name="c", subcore_axis_name="s")

    @functools.partial(
        pl.kernel,
        mesh=mesh,
        out_type=[
            jax.ShapeDtypeStruct((B, D), jnp.float32),
            jax.ShapeDtypeStruct((B, D), jnp.float32),
        ],
        scratch_types=[
            pltpu.VMEM((PER_W,), jnp.int32),
            pltpu.VMEM((CHUNK, D), jnp.float32),
            pltpu.VMEM((CHUNK, D), jnp.float32),
            pltpu.SemaphoreType.DMA,
        ],
        compiler_params=pltpu.CompilerParams(use_tc_tiling_on_sc=False),
    )
    def k(x_hbm, tr_hbm, ti_hbm, outr_hbm, outi_hbm, idx_v, buf_r, buf_i, sem):
        wid = lax.axis_index("s") * NC + lax.axis_index("c")
        pltpu.sync_copy(x_hbm.at[pl.ds(wid * PER_W, PER_W)], idx_v)

        def body(j, _):
            idx_row = idx_v.at[pl.ds(j * CHUNK, CHUNK)]
            base = wid * PER_W + j * CHUNK
            cr = pltpu.async_copy(tr_hbm.at[idx_row], buf_r, sem)
            ci = pltpu.async_copy(ti_hbm.at[idx_row], buf_i, sem)
            cr.wait()
            ci.wait()
            pltpu.sync_copy(buf_r, outr_hbm.at[pl.ds(base, CHUNK)])
            pltpu.sync_copy(buf_i, outi_hbm.at[pl.ds(base, CHUNK)])
            return _

        lax.fori_loop(0, N_CHUNKS, body, 0, unroll=False)

    return k(x1d, table_real, table_imag)


def kernel(x, table_real, table_imag):
    x1d = x.reshape(B).astype(jnp.int32)
    out_r, out_i = _sc_gather(x1d, table_real, table_imag)
    out_c = lax.complex(out_r, out_i)
    return out_c.reshape(BATCH, SEQ, D)


# speedup vs baseline: 1.8337x; 1.3407x over previous
import functools

import jax
import jax.numpy as jnp
from jax import lax
from jax.experimental import pallas as pl
from jax.experimental.pallas import tpu as pltpu
from jax.experimental.pallas import tpu_sc as plsc

VOCAB = 100000
D = 64
BATCH = 4096
SEQ = 50

_INFO = plsc.get_sparse_core_info()
NC = _INFO.num_cores
NS = _INFO.num_subcores
NW = NC * NS

ITEMS_PER_W = BATCH // NW
PAIR = 2
N_CHUNKS = ITEMS_PER_W // PAIR


def _sc_gather(x2d, table_real, table_imag):
    mesh = plsc.VectorSubcoreMesh(core_axis_name="c", subcore_axis_name="s")

    @functools.partial(
        pl.kernel,
        mesh=mesh,
        out_type=[
            jax.ShapeDtypeStruct((BATCH, SEQ, D), jnp.float32),
            jax.ShapeDtypeStruct((BATCH, SEQ, D), jnp.float32),
        ],
        scratch_types=[
            pltpu.VMEM((N_CHUNKS, PAIR * SEQ), jnp.int32),
            pltpu.VMEM((PAIR * SEQ, D), jnp.float32),
            pltpu.VMEM((PAIR * SEQ, D), jnp.float32),
            pltpu.SemaphoreType.DMA,
        ],
        compiler_params=pltpu.CompilerParams(use_tc_tiling_on_sc=False),
    )
    def k(x_hbm, tr_hbm, ti_hbm, outr_hbm, outi_hbm, idx_v, buf_r, buf_i, sem):
        wid = lax.axis_index("s") * NC + lax.axis_index("c")
        pltpu.sync_copy(x_hbm.at[pl.ds(wid * N_CHUNKS, N_CHUNKS)], idx_v)

        def body(j, _):
            idx_row = idx_v.at[j]
            item = wid * ITEMS_PER_W + j * PAIR
            cr = pltpu.async_copy(tr_hbm.at[idx_row], buf_r, sem)
            ci = pltpu.async_copy(ti_hbm.at[idx_row], buf_i, sem)
            cr.wait()
            ci.wait()
            pltpu.sync_copy(buf_r.at[pl.ds(0, SEQ)], outr_hbm.at[item])
            pltpu.sync_copy(buf_r.at[pl.ds(SEQ, SEQ)], outr_hbm.at[item + 1])
            pltpu.sync_copy(buf_i.at[pl.ds(0, SEQ)], outi_hbm.at[item])
            pltpu.sync_copy(buf_i.at[pl.ds(SEQ, SEQ)], outi_hbm.at[item + 1])
            return _

        lax.fori_loop(0, N_CHUNKS, body, 0, unroll=False)

    return k(x2d, table_real, table_imag)


def kernel(x, table_real, table_imag):
    x2d = x.reshape(BATCH // PAIR, PAIR * SEQ).astype(jnp.int32)
    out_r, out_i = _sc_gather(x2d, table_real, table_imag)
    return lax.complex(out_r, out_i)


# speedup vs baseline: 2.7668x; 1.5089x over previous
import functools

import jax
import jax.numpy as jnp
from jax import lax
from jax.experimental import pallas as pl
from jax.experimental.pallas import tpu as pltpu
from jax.experimental.pallas import tpu_sc as plsc

VOCAB = 100000
D = 64
BATCH = 4096
SEQ = 50

_INFO = plsc.get_sparse_core_info()
NC = _INFO.num_cores
NS = _INFO.num_subcores
NW = NC * NS

BPW = BATCH // NW
LANES = 16


def _sc_gather(xT, table_real, table_imag):
    mesh = plsc.VectorSubcoreMesh(core_axis_name="c", subcore_axis_name="s")

    @functools.partial(
        pl.kernel,
        mesh=mesh,
        out_type=[
            jax.ShapeDtypeStruct((SEQ, D // 8, BATCH // BPW, 8, BPW), jnp.float32),
            jax.ShapeDtypeStruct((SEQ, D // 8, BATCH // BPW, 8, BPW), jnp.float32),
        ],
        scratch_types=[
            pltpu.VMEM((SEQ, BPW), jnp.int32),
            pltpu.VMEM((BPW, D), jnp.float32),
            pltpu.VMEM((BPW, D), jnp.float32),
            pltpu.VMEM((D // 8, 8, BPW), jnp.float32),
            pltpu.VMEM((D // 8, 8, BPW), jnp.float32),
            pltpu.SemaphoreType.DMA,
        ],
        compiler_params=pltpu.CompilerParams(
            use_tc_tiling_on_sc=False, needs_layout_passes=False),
    )
    def k(xT_hbm, tr_hbm, ti_hbm, outr_hbm, outi_hbm,
          idx_v, buf_r, buf_i, bufT_r, bufT_i, sem):
        wid = lax.axis_index("s") * NC + lax.axis_index("c")
        pltpu.sync_copy(xT_hbm.at[:, pl.ds(wid * BPW, BPW)], idx_v)
        iota = lax.iota(jnp.int32, LANES)

        def body(s, _):
            idx_row = idx_v.at[s]
            cr = pltpu.async_copy(tr_hbm.at[idx_row], buf_r, sem)
            ci = pltpu.async_copy(ti_hbm.at[idx_row], buf_i, sem)
            cr.wait()
            ci.wait()

            def tgroup(tg, _c):
                t_idx = tg * LANES + iota
                for dh in range(D // 8):
                    for dl in range(8):
                        d_idx = jnp.full((LANES,), dh * 8 + dl, jnp.int32)
                        vr = plsc.load_gather(buf_r, [t_idx, d_idx])
                        vi = plsc.load_gather(buf_i, [t_idx, d_idx])
                        bufT_r[dh, dl, pl.ds(tg * LANES, LANES)] = vr
                        bufT_i[dh, dl, pl.ds(tg * LANES, LANES)] = vi
                return _c

            lax.fori_loop(0, BPW // LANES, tgroup, 0, unroll=False)
            pltpu.sync_copy(bufT_r, outr_hbm.at[s, :, wid])
            pltpu.sync_copy(bufT_i, outi_hbm.at[s, :, wid])
            return _

        lax.fori_loop(0, SEQ, body, 0, unroll=False)

    return k(xT, table_real, table_imag)


def kernel(x, table_real, table_imag):
    xT = jnp.transpose(x.astype(jnp.int32), (1, 0))
    out_r, out_i = _sc_gather(xT, table_real, table_imag)

    def unfold(o):
        return jnp.transpose(o, (2, 4, 0, 1, 3)).reshape(BATCH, SEQ, D)

    return lax.complex(unfold(out_r), unfold(out_i))


# speedup vs baseline: 3.3865x; 1.2240x over previous
import functools

import jax
import jax.numpy as jnp
from jax import lax
from jax.experimental import pallas as pl
from jax.experimental.pallas import tpu as pltpu
from jax.experimental.pallas import tpu_sc as plsc

VOCAB = 100000
D = 64
BATCH = 4096
SEQ = 50

_INFO = plsc.get_sparse_core_info()
NC = _INFO.num_cores
NS = _INFO.num_subcores
NW = NC * NS

BPW = BATCH // NW
LANES = 16
TP = BPW + 9


def _sc_gather(xT, table_real, table_imag):
    mesh = plsc.VectorSubcoreMesh(core_axis_name="c", subcore_axis_name="s")

    @functools.partial(
        pl.kernel,
        mesh=mesh,
        out_type=[
            jax.ShapeDtypeStruct((SEQ, D // 8, BATCH // BPW, 8, BPW), jnp.float32),
            jax.ShapeDtypeStruct((SEQ, D // 8, BATCH // BPW, 8, BPW), jnp.float32),
        ],
        scratch_types=[
            pltpu.VMEM((SEQ, BPW), jnp.int32),
            pltpu.VMEM((BPW, D), jnp.float32),
            pltpu.VMEM((BPW, D), jnp.float32),
            pltpu.VMEM((D // 8, 8, TP), jnp.float32),
            pltpu.VMEM((D // 8, 8, TP), jnp.float32),
            pltpu.SemaphoreType.DMA,
        ],
        compiler_params=pltpu.CompilerParams(
            use_tc_tiling_on_sc=False, needs_layout_passes=False),
    )
    def k(xT_hbm, tr_hbm, ti_hbm, outr_hbm, outi_hbm,
          idx_v, buf_r, buf_i, bufT_r, bufT_i, sem):
        wid = lax.axis_index("s") * NC + lax.axis_index("c")
        pltpu.sync_copy(xT_hbm.at[:, pl.ds(wid * BPW, BPW)], idx_v)
        iota = lax.iota(jnp.int32, LANES)
        dhs = [((16 * g + iota) >> 3).astype(jnp.int32) for g in range(D // LANES)]
        dls = [((16 * g + iota) & 7).astype(jnp.int32) for g in range(D // LANES)]

        def body(s, _):
            idx_row = idx_v.at[s]
            cr = pltpu.async_copy(tr_hbm.at[idx_row], buf_r, sem)
            ci = pltpu.async_copy(ti_hbm.at[idx_row], buf_i, sem)
            cr.wait()
            ci.wait()

            def token(t, _c):
                t_idx = jnp.full((LANES,), t, jnp.int32)
                for g in range(D // LANES):
                    vr = buf_r[t, pl.ds(g * LANES, LANES)]
                    vi = buf_i[t, pl.ds(g * LANES, LANES)]
                    plsc.store_scatter(bufT_r, [dhs[g], dls[g], t_idx], vr)
                    plsc.store_scatter(bufT_i, [dhs[g], dls[g], t_idx], vi)
                return _c

            lax.fori_loop(0, BPW, token, 0, unroll=False)
            pltpu.sync_copy(bufT_r.at[:, :, pl.ds(0, BPW)], outr_hbm.at[s, :, wid])
            pltpu.sync_copy(bufT_i.at[:, :, pl.ds(0, BPW)], outi_hbm.at[s, :, wid])
            return _

        lax.fori_loop(0, SEQ, body, 0, unroll=False)

    return k(xT, table_real, table_imag)


def kernel(x, table_real, table_imag):
    xT = jnp.transpose(x.astype(jnp.int32), (1, 0))
    out_r, out_i = _sc_gather(xT, table_real, table_imag)

    def unfold(o):
        return jnp.transpose(o, (2, 4, 0, 1, 3)).reshape(BATCH, SEQ, D)

    return lax.complex(unfold(out_r), unfold(out_i))


# speedup vs baseline: 3.6218x; 1.0695x over previous
import functools

import jax
import jax.numpy as jnp
from jax import lax
from jax.experimental import pallas as pl
from jax.experimental.pallas import tpu as pltpu
from jax.experimental.pallas import tpu_sc as plsc

VOCAB = 100000
D = 64
BATCH = 4096
SEQ = 50

_INFO = plsc.get_sparse_core_info()
NC = _INFO.num_cores
NS = _INFO.num_subcores
NW = NC * NS

BPW = BATCH // NW
LANES = 16
TP = BPW + 9


def _sc_gather(xT, table_real, table_imag):
    mesh = plsc.VectorSubcoreMesh(core_axis_name="c", subcore_axis_name="s")

    @functools.partial(
        pl.kernel,
        mesh=mesh,
        out_type=[
            jax.ShapeDtypeStruct((SEQ, D // 8, BATCH // BPW, 8, BPW), jnp.float32),
            jax.ShapeDtypeStruct((SEQ, D // 8, BATCH // BPW, 8, BPW), jnp.float32),
        ],
        scratch_types=[
            pltpu.VMEM((SEQ, BPW), jnp.int32),
            pltpu.VMEM((2, BPW, D), jnp.float32),
            pltpu.VMEM((2, BPW, D), jnp.float32),
            pltpu.VMEM((2, D // 8, 8, TP), jnp.float32),
            pltpu.VMEM((2, D // 8, 8, TP), jnp.float32),
            pltpu.SemaphoreType.DMA,
            pltpu.SemaphoreType.DMA,
            pltpu.SemaphoreType.DMA,
            pltpu.SemaphoreType.DMA,
        ],
        compiler_params=pltpu.CompilerParams(
            use_tc_tiling_on_sc=False, needs_layout_passes=False),
    )
    def k(xT_hbm, tr_hbm, ti_hbm, outr_hbm, outi_hbm,
          idx_v, buf_r, buf_i, bufT_r, bufT_i, semg0, semg1, semw0, semw1):
        wid = lax.axis_index("s") * NC + lax.axis_index("c")
        semg = [semg0, semg1]
        semw = [semw0, semw1]
        pltpu.sync_copy(xT_hbm.at[:, pl.ds(wid * BPW, BPW)], idx_v)
        iota = lax.iota(jnp.int32, LANES)
        dhs = [((16 * g + iota) >> 3).astype(jnp.int32) for g in range(D // LANES)]
        dls = [((16 * g + iota) & 7).astype(jnp.int32) for g in range(D // LANES)]

        def g_copies(s, b):
            idx_row = idx_v.at[s]
            return (pltpu.make_async_copy(tr_hbm.at[idx_row], buf_r.at[b], semg[b]),
                    pltpu.make_async_copy(ti_hbm.at[idx_row], buf_i.at[b], semg[b]))

        def w_copies(s, b):
            return (pltpu.make_async_copy(bufT_r.at[b, :, :, pl.ds(0, BPW)],
                                          outr_hbm.at[s, :, wid], semw[b]),
                    pltpu.make_async_copy(bufT_i.at[b, :, :, pl.ds(0, BPW)],
                                          outi_hbm.at[s, :, wid], semw[b]))

        def g_start(s, b):
            for c in g_copies(s, b):
                c.start()

        def g_wait(s, b):
            for c in g_copies(s, b):
                c.wait()

        def w_start(s, b):
            for c in w_copies(s, b):
                c.start()

        def w_wait(s, b):
            for c in w_copies(s, b):
                c.wait()

        g_start(0, 0)

        def rnd(r, _):
            for b in range(2):
                s = 2 * r + b

                @pl.when(s + 1 < SEQ)
                def _pref():
                    g_start(s + 1, 1 - b)

                @pl.when(s >= 2)
                def _drain():
                    w_wait(s - 2, b)

                g_wait(s, b)

                def token(t, _c):
                    t_idx = jnp.full((LANES,), t, jnp.int32)
                    for g in range(D // LANES):
                        vr = buf_r[b, t, pl.ds(g * LANES, LANES)]
                        vi = buf_i[b, t, pl.ds(g * LANES, LANES)]
                        plsc.store_scatter(bufT_r.at[b], [dhs[g], dls[g], t_idx], vr)
                        plsc.store_scatter(bufT_i.at[b], [dhs[g], dls[g], t_idx], vi)
                    return _c

                lax.fori_loop(0, BPW, token, 0, unroll=2)
                w_start(s, b)
            return _

        lax.fori_loop(0, SEQ // 2, rnd, 0, unroll=False)
        w_wait(SEQ - 2, 0)
        w_wait(SEQ - 1, 1)

    return k(xT, table_real, table_imag)


def kernel(x, table_real, table_imag):
    xT = jnp.transpose(x.astype(jnp.int32), (1, 0))
    out_r, out_i = _sc_gather(xT, table_real, table_imag)

    def unfold(o):
        return jnp.transpose(o, (2, 4, 0, 1, 3)).reshape(BATCH, SEQ, D)

    return lax.complex(unfold(out_r), unfold(out_i))
